# Initial kernel scaffold; baseline (speedup 1.0000x reference)
import jax
import jax.numpy as jnp
from jax.experimental import pallas as pl


def kernel(hidden_states, w1, w2, topk_weights, topk_ids):
    raise NotImplementedError("write your pallas kernel here")


# speedup vs baseline: 1.2384x; 1.2384x over previous
import functools

import jax
import jax.numpy as jnp
from jax.experimental import pallas as pl


def _moe_step(ids_ref, wts_ref, x_ref, w1_ref, w2_ref, out_ref, *, N):
    e = pl.program_id(0)
    x = x_ref[...]
    w1 = w1_ref[0]
    h = jax.lax.dot_general(
        x, w1, (((1,), (1,)), ((), ())), preferred_element_type=jnp.float32
    )
    gate = h[:, :N]
    up = h[:, N:]
    act = gate * jax.lax.logistic(gate) * up
    w2 = w2_ref[0]
    eo = jax.lax.dot_general(
        act, w2, (((1,), (1,)), ((), ())), preferred_element_type=jnp.float32
    )
    sel = jnp.where(ids_ref[...] == e, wts_ref[...], 0.0)
    wpe = jnp.sum(sel, axis=1)
    contrib = eo * wpe[:, None]

    @pl.when(e == 0)
    def _init():
        out_ref[...] = contrib

    @pl.when(e != 0)
    def _acc():
        out_ref[...] += contrib


def kernel(hidden_states, w1, w2, topk_weights, topk_ids):
    M, K = hidden_states.shape
    E, twoN, _ = w1.shape
    N = twoN // 2
    grid = (E,)
    out = pl.pallas_call(
        functools.partial(_moe_step, N=N),
        grid=grid,
        in_specs=[
            pl.BlockSpec(topk_ids.shape, lambda e: (0, 0)),
            pl.BlockSpec(topk_weights.shape, lambda e: (0, 0)),
            pl.BlockSpec((M, K), lambda e: (0, 0)),
            pl.BlockSpec((1, twoN, K), lambda e: (e, 0, 0)),
            pl.BlockSpec((1, K, N), lambda e: (e, 0, 0)),
        ],
        out_specs=pl.BlockSpec((M, K), lambda e: (0, 0)),
        out_shape=jax.ShapeDtypeStruct((M, K), hidden_states.dtype),
    )(topk_ids, topk_weights, hidden_states, w1, w2)
    return out


# speedup vs baseline: 1.4130x; 1.1411x over previous
import functools

import jax
import jax.numpy as jnp
from jax.experimental import pallas as pl

_G = 2


def _moe_step(ids_ref, wts_ref, x_ref, w1_ref, w2_ref, out_ref, *, N, G):
    i = pl.program_id(0)
    x = x_ref[...]
    ids = ids_ref[...]
    wts = wts_ref[...]
    contrib = None
    for g in range(G):
        w1 = w1_ref[g]
        h = jax.lax.dot_general(
            x, w1, (((1,), (1,)), ((), ())),
            preferred_element_type=jnp.float32,
        )
        gate = h[:, :N]
        up = h[:, N:]
        act = gate * jax.lax.logistic(gate) * up
        e = i * G + g
        wpe = jnp.sum(jnp.where(ids == e, wts, 0.0), axis=1)
        act = act * wpe[:, None]
        w2 = w2_ref[g]
        c = jax.lax.dot_general(
            act, w2, (((1,), (1,)), ((), ())),
            preferred_element_type=jnp.float32,
        )
        contrib = c if contrib is None else contrib + c

    @pl.when(i == 0)
    def _init():
        out_ref[...] = contrib

    @pl.when(i != 0)
    def _acc():
        out_ref[...] += contrib


def kernel(hidden_states, w1, w2, topk_weights, topk_ids):
    M, K = hidden_states.shape
    E, twoN, _ = w1.shape
    N = twoN // 2
    G = _G
    grid = (E // G,)
    out = pl.pallas_call(
        functools.partial(_moe_step, N=N, G=G),
        grid=grid,
        in_specs=[
            pl.BlockSpec(topk_ids.shape, lambda i: (0, 0)),
            pl.BlockSpec(topk_weights.shape, lambda i: (0, 0)),
            pl.BlockSpec((M, K), lambda i: (0, 0)),
            pl.BlockSpec((G, twoN, K), lambda i: (i, 0, 0)),
            pl.BlockSpec((G, K, N), lambda i: (i, 0, 0)),
        ],
        out_specs=pl.BlockSpec((M, K), lambda i: (0, 0)),
        out_shape=jax.ShapeDtypeStruct((M, K), hidden_states.dtype),
    )(topk_ids, topk_weights, hidden_states, w1, w2)
    return out
